# Initial kernel scaffold; baseline (speedup 1.0000x reference)
import jax
import jax.numpy as jnp
from jax.experimental import pallas as pl


def kernel(x, edge_index, W0, b0, W1, b1):
    raise NotImplementedError("write your pallas kernel here")


# speedup vs baseline: 5.2614x; 5.2614x over previous
import functools

import jax
import jax.numpy as jnp
from jax import lax
from jax.experimental import pallas as pl
from jax.experimental.pallas import tpu as pltpu
from jax.experimental.pallas import tpu_sc as plsc

N = 10000
E = 320000
D = 128
DH = 64

NC = 2
NS = 16
NW = NC * NS

CHUNK = 128
EPT = ((E // NW + CHUNK - 1) // CHUNK) * CHUNK
EPAD = EPT * NW
NPAD = 10112
ZROWS = NPAD // NS


def _sc_segment_sum_body(y_hbm, src_hbm, dst_hbm, zeros_hbm, part_hbm,
                         sidx, didx, rows, accum, sem):
    c = lax.axis_index("c")
    s = lax.axis_index("s")
    wid = s * NC + c

    z0 = s * ZROWS
    pltpu.sync_copy(zeros_hbm.at[pl.ds(z0, ZROWS)], accum.at[pl.ds(z0, ZROWS)])
    plsc.subcore_barrier()

    def chunk_body(j, carry):
        base = wid * EPT + j * CHUNK
        pltpu.sync_copy(src_hbm.at[pl.ds(base, CHUNK)], sidx)
        pltpu.sync_copy(dst_hbm.at[pl.ds(base, CHUNK)], didx)
        pltpu.async_copy(y_hbm.at[sidx], rows, sem).wait()
        pltpu.sync_copy(rows, accum.at[didx], add=True)
        return carry

    lax.fori_loop(0, EPT // CHUNK, chunk_body, 0)
    plsc.subcore_barrier()

    pltpu.sync_copy(accum.at[pl.ds(z0, ZROWS)], part_hbm.at[c, pl.ds(z0, ZROWS)])


_sc_segment_sum = pl.kernel(
    _sc_segment_sum_body,
    out_type=jax.ShapeDtypeStruct((NC, NPAD, DH), jnp.float32),
    mesh=plsc.VectorSubcoreMesh(
        core_axis_name="c", subcore_axis_name="s", num_cores=NC, num_subcores=NS
    ),
    scratch_types=[
        pltpu.VMEM((CHUNK,), jnp.int32),
        pltpu.VMEM((CHUNK,), jnp.int32),
        pltpu.VMEM((CHUNK, DH), jnp.float32),
        pltpu.VMEM_SHARED((NPAD, DH), jnp.float32),
        pltpu.SemaphoreType.DMA,
    ],
    compiler_params=pltpu.CompilerParams(use_tc_tiling_on_sc=False),
)


def _tc_dense_body(part_ref, xp_ref, w_ref, b_ref, o_ref):
    agg = part_ref[0, :N] + part_ref[1, :N]
    h = jnp.dot(agg, w_ref[...], preferred_element_type=jnp.float32)
    o_ref[...] = xp_ref[...] + jnp.tanh(h + b_ref[...])


def _tc_dense(part, x_part, w, b):
    return pl.pallas_call(
        _tc_dense_body,
        out_shape=jax.ShapeDtypeStruct((N, DH), jnp.float32),
    )(part, x_part, w, b.reshape(1, DH))


@jax.jit
def kernel(x, edge_index, W0, b0, W1, b1):
    x0 = x[:, :DH]
    x1 = x[:, DH:]
    pad = EPAD - E
    src = jnp.concatenate([edge_index[0], jnp.zeros((pad,), jnp.int32)])
    dst = jnp.concatenate([edge_index[1], jnp.full((pad,), N, jnp.int32)])
    zeros = jnp.zeros((NPAD, DH), jnp.float32)

    p0 = _sc_segment_sum(x1, src, dst, zeros)
    y0 = _tc_dense(p0, x0, W0, b0)
    p1 = _sc_segment_sum(y0, src, dst, zeros)
    y1 = _tc_dense(p1, x1, W1, b1)
    return jnp.concatenate([y0, y1], axis=-1)
